# baseline (speedup 1.0000x reference)
import functools

import jax
import jax.numpy as jnp
import numpy as np
from jax import lax
from jax.experimental import pallas as pl
from jax.experimental.pallas import tpu as pltpu
from jax.experimental.pallas import tpu_sc as plsc

N = 100000
E = 3200000
RADIUS = 0.015
INV_R = np.float32(1.0 / RADIUS)

NC = 2
NS = 16
NW = NC * NS
TE = E // NW
C = 4000
NCH = TE // C
NCHN = N // C
STAGE_IT = (NCHN + NS - 1) // NS

BN = 2000
NB = N // BN
WB = BN
NWB = N // WB
WB_IT = (NWB + NS - 1) // NS
VEL_MEAN = np.float32(0.0)
VEL_STD = np.float32(1.0)


def _rsqrt_newton(s):
  i = lax.bitcast_convert_type(s, jnp.int32)
  i = jnp.int32(0x5F3759DF) - lax.shift_right_logical(i, 1)
  y = lax.bitcast_convert_type(i, jnp.float32)
  for _ in range(3):
    y = y * (jnp.float32(1.5) - jnp.float32(0.5) * s * y * y)
  return y


def _sc_body(xs, ys, zs, snd, rcv, out,
             shx, shy, shz, accx, accy, accz, accn,
             sidx, ridx, sx, sy, sz, rx, ry, rz, zbuf):
  cid = lax.axis_index("c")
  sid = lax.axis_index("s")
  wid = cid * NS + sid

  def zb(i, _):
    zbuf[pl.ds(i * 16, 16)] = jnp.zeros((16,), jnp.float32)
    return 0
  lax.fori_loop(0, C // 16, zb, 0)

  for src, dst in ((xs, shx), (ys, shy), (zs, shz)):
    def stage(k, _, src=src, dst=dst):
      c = sid + NS * k
      @pl.when(c < NCHN)
      def _():
        pltpu.sync_copy(src.at[pl.ds(c * C, C)], sx)
        pltpu.sync_copy(sx, dst.at[pl.ds(c * C, C)])
      return 0
    lax.fori_loop(0, STAGE_IT, stage, 0)
  for acc in (accx, accy, accz, accn):
    def zero(k, _, acc=acc):
      c = sid + NS * k
      @pl.when(c < NCHN)
      def _():
        pltpu.sync_copy(zbuf, acc.at[pl.ds(c * C, C)])
      return 0
    lax.fori_loop(0, STAGE_IT, zero, 0)
  plsc.subcore_barrier()

  def chunk(g, _):
    base = wid * TE + g * C
    pltpu.sync_copy(snd.at[pl.ds(base, C)], sidx)
    pltpu.sync_copy(rcv.at[pl.ds(base, C)], ridx)
    pltpu.sync_copy(shx.at[sidx], sx)
    pltpu.sync_copy(shy.at[sidx], sy)
    pltpu.sync_copy(shz.at[sidx], sz)
    pltpu.sync_copy(shx.at[ridx], rx)
    pltpu.sync_copy(shy.at[ridx], ry)
    pltpu.sync_copy(shz.at[ridx], rz)

    def compute(i, _):
      s = pl.ds(i * 16, 16)
      vdx = (sx[s] - rx[s]) * INV_R
      vdy = (sy[s] - ry[s]) * INV_R
      vdz = (sz[s] - rz[s]) * INV_R
      ss = vdx * vdx + vdy * vdy + vdz * vdz
      vn = ss * _rsqrt_newton(ss)
      sx[s] = vdx
      sy[s] = vdy
      sz[s] = vdz
      rx[s] = vn
      return 0
    lax.fori_loop(0, C // 16, compute, 0)

    pltpu.sync_copy(sx, accx.at[ridx], add=True)
    pltpu.sync_copy(sy, accy.at[ridx], add=True)
    pltpu.sync_copy(sz, accz.at[ridx], add=True)
    pltpu.sync_copy(rx, accn.at[ridx], add=True)
    return 0
  lax.fori_loop(0, NCH, chunk, 0)
  plsc.subcore_barrier()

  for comp, acc in enumerate((accx, accy, accz, accn)):
    def wb(k, _, comp=comp, acc=acc):
      c = sid + NS * k
      @pl.when(c < NWB)
      def _():
        pltpu.sync_copy(acc.at[pl.ds(c * WB, WB)], sx.at[pl.ds(0, WB)])
        pltpu.sync_copy(sx.at[pl.ds(0, WB)],
                        out.at[pl.ds(c * 8 * WB + (cid * 4 + comp) * WB, WB)])
      return 0
    lax.fori_loop(0, WB_IT, wb, 0)


@functools.cache
def _sc_edge_kernel():
  return pl.kernel(
    _sc_body,
    out_type=jax.ShapeDtypeStruct((NWB * 2 * 4 * WB,), jnp.float32),
    mesh=plsc.VectorSubcoreMesh(core_axis_name="c", subcore_axis_name="s",
                                num_cores=NC, num_subcores=NS),
    scratch_types=[
        pltpu.VMEM_SHARED((N,), jnp.float32),
        pltpu.VMEM_SHARED((N,), jnp.float32),
        pltpu.VMEM_SHARED((N,), jnp.float32),
        pltpu.VMEM_SHARED((N,), jnp.float32),
        pltpu.VMEM_SHARED((N,), jnp.float32),
        pltpu.VMEM_SHARED((N,), jnp.float32),
        pltpu.VMEM_SHARED((N,), jnp.float32),
        pltpu.VMEM((C,), jnp.int32),
        pltpu.VMEM((C,), jnp.int32),
        pltpu.VMEM((C,), jnp.float32),
        pltpu.VMEM((C,), jnp.float32),
        pltpu.VMEM((C,), jnp.float32),
        pltpu.VMEM((C,), jnp.float32),
        pltpu.VMEM((C,), jnp.float32),
        pltpu.VMEM((C,), jnp.float32),
        pltpu.VMEM((C,), jnp.float32),
    ],
  )

def _tc_a_body(pos_ref, emb_ref, pt_ref, out_ref):
  pos = pos_ref[...]
  emb = emb_ref[...]
  pt = pt_ref[...]

  mrp = pos[:, 15:18]
  vel = pos[:, 3:18] - pos[:, 0:15]
  nvel = (vel - VEL_MEAN) / VEL_STD
  relp = pos[:, 0:15] - jnp.concatenate([mrp] * 5, axis=1)
  db = jnp.concatenate([mrp - jnp.float32(0.1), jnp.float32(0.9) - mrp],
                       axis=1)
  ncdb = jnp.clip(db * INV_R, -1.0, 1.0)

  oh = (pt == lax.broadcasted_iota(jnp.int32, (BN, 9), 1)).astype(jnp.float32)
  temb = jnp.dot(oh, emb, preferred_element_type=jnp.float32)

  out_ref[...] = jnp.concatenate([nvel, relp, ncdb, temb], axis=1)


_tc_a_kernel = pl.pallas_call(
    _tc_a_body,
    grid=(NB,),
    in_specs=[
        pl.BlockSpec((BN, 18), lambda i: (i, 0)),
        pl.BlockSpec((9, 16), lambda i: (0, 0)),
        pl.BlockSpec((BN, 1), lambda i: (i, 0)),
    ],
    out_specs=pl.BlockSpec((BN, 52), lambda i: (i, 0)),
    out_shape=jax.ShapeDtypeStruct((N, 52), jnp.float32),
)


def _tc_b_body(pos_ref, agg_ref, a_ref, out_ref):
  pT = jnp.transpose(pos_ref[...][:, 9:18])
  pr = agg_ref[0]

  v = pT[6:9] - pT[3:6]
  vp = pT[3:6] - pT[0:3]
  a = v - vp
  eps = jnp.float32(1e-8)
  e1 = v / (jnp.sqrt(jnp.sum(v * v, axis=0, keepdims=True)) + eps)
  u2 = a - jnp.sum(e1 * a, axis=0, keepdims=True) * e1
  e2 = u2 / (jnp.sqrt(jnp.sum(u2 * u2, axis=0, keepdims=True)) + eps)
  e3 = jnp.concatenate([
      e1[1:2] * e2[2:3] - e1[2:3] * e2[1:2],
      e1[2:3] * e2[0:1] - e1[0:1] * e2[2:3],
      e1[0:1] * e2[1:2] - e1[1:2] * e2[0:1],
  ], axis=0)
  rT = jnp.concatenate([
      e1[0:1], e2[0:1], e3[0:1],
      e1[1:2], e2[1:2], e3[1:2],
      e1[2:3], e2[2:3], e3[2:3],
  ], axis=0)

  big = jnp.concatenate([pr, rT], axis=0)
  r = lax.broadcasted_iota(jnp.int32, (17, 13), 0)
  c = lax.broadcasted_iota(jnp.int32, (17, 13), 1)
  m = (((r < 8) & (c < 4) & (r % 4 == c))
       | ((r >= 8) & (c >= 4) & (r - 8 == c - 4))).astype(jnp.float32)
  stripe = lax.dot_general(big, m,
                           dimension_numbers=(((0,), (0,)), ((), ())),
                           preferred_element_type=jnp.float32)
  out_ref[...] = jnp.concatenate([a_ref[...], stripe], axis=1)


_tc_b_kernel = pl.pallas_call(
    _tc_b_body,
    grid=(NB,),
    in_specs=[
        pl.BlockSpec((BN, 18), lambda i: (i, 0)),
        pl.BlockSpec((1, 8, BN), lambda i: (i, 0, 0)),
        pl.BlockSpec((BN, 52), lambda i: (i, 0)),
    ],
    out_specs=pl.BlockSpec((BN, 65), lambda i: (i, 0)),
    out_shape=jax.ShapeDtypeStruct((N, 65), jnp.float32),
)


def kernel(position_sequence, type_embedding, particle_types, edge_index):
  mrp = position_sequence[:, -1]
  xs = mrp[:, 0]
  ys = mrp[:, 1]
  zs = mrp[:, 2]
  partials = _sc_edge_kernel()(xs, ys, zs, edge_index[0], edge_index[1])
  partials = partials.reshape(NWB, 8, WB)
  pos18 = position_sequence.reshape(N, 18)
  out_a = _tc_a_kernel(pos18, type_embedding, particle_types.reshape(N, 1))
  return _tc_b_kernel(pos18, partials, out_a)


# speedup vs baseline: 1.0336x; 1.0336x over previous
import functools

import jax
import jax.numpy as jnp
import numpy as np
from jax import lax
from jax.experimental import pallas as pl
from jax.experimental.pallas import tpu as pltpu
from jax.experimental.pallas import tpu_sc as plsc

N = 100000
E = 3200000
RADIUS = 0.015
INV_R = np.float32(1.0 / RADIUS)

NC = 2
NS = 16
NW = NC * NS
TE = E // NW
C = 4000
NCH = TE // C
NCHN = N // C
STAGE_IT = (NCHN + NS - 1) // NS

BN = 2000
NB = N // BN
WB = BN
NWB = N // WB
WB_IT = (NWB + NS - 1) // NS
VEL_MEAN = np.float32(0.0)
VEL_STD = np.float32(1.0)


def _rsqrt_newton(s):
  i = lax.bitcast_convert_type(s, jnp.int32)
  i = jnp.int32(0x5F3759DF) - lax.shift_right_logical(i, 1)
  y = lax.bitcast_convert_type(i, jnp.float32)
  for _ in range(3):
    y = y * (jnp.float32(1.5) - jnp.float32(0.5) * s * y * y)
  return y


def _sc_body(xs, ys, zs, snd, rcv, out,
             shx, shy, shz, accx, accy, accz, accn,
             sidx, ridx, sx, sy, sz, rx, ry, rz, zbuf):
  cid = lax.axis_index("c")
  sid = lax.axis_index("s")
  wid = cid * NS + sid

  def zb(i, _):
    zbuf[pl.ds(i * 16, 16)] = jnp.zeros((16,), jnp.float32)
    return 0
  lax.fori_loop(0, C // 16, zb, 0)

  for src, dst in ((xs, shx), (ys, shy), (zs, shz)):
    def stage(k, _, src=src, dst=dst):
      c = sid + NS * k
      @pl.when(c < NCHN)
      def _():
        pltpu.sync_copy(src.at[pl.ds(c * C, C)], sx)
        pltpu.sync_copy(sx, dst.at[pl.ds(c * C, C)])
      return 0
    lax.fori_loop(0, STAGE_IT, stage, 0)
  for acc in (accx, accy, accz, accn):
    def zero(k, _, acc=acc):
      c = sid + NS * k
      @pl.when(c < NCHN)
      def _():
        pltpu.sync_copy(zbuf, acc.at[pl.ds(c * C, C)])
      return 0
    lax.fori_loop(0, STAGE_IT, zero, 0)
  plsc.subcore_barrier()

  def chunk(g, _):
    base = wid * TE + g * C
    pltpu.sync_copy(snd.at[pl.ds(base, C)], sidx)
    pltpu.sync_copy(rcv.at[pl.ds(base, C)], ridx)
    pltpu.sync_copy(shx.at[sidx], sx)
    pltpu.sync_copy(shy.at[sidx], sy)
    pltpu.sync_copy(shz.at[sidx], sz)
    pltpu.sync_copy(shx.at[ridx], rx)
    pltpu.sync_copy(shy.at[ridx], ry)
    pltpu.sync_copy(shz.at[ridx], rz)

    def compute(i, _):
      s = pl.ds(i * 16, 16)
      vdx = (sx[s] - rx[s]) * INV_R
      vdy = (sy[s] - ry[s]) * INV_R
      vdz = (sz[s] - rz[s]) * INV_R
      ss = vdx * vdx + vdy * vdy + vdz * vdz
      vn = ss * _rsqrt_newton(ss)
      sx[s] = vdx
      sy[s] = vdy
      sz[s] = vdz
      rx[s] = vn
      return 0
    lax.fori_loop(0, C // 16, compute, 0)

    pltpu.sync_copy(sx, accx.at[ridx], add=True)
    pltpu.sync_copy(sy, accy.at[ridx], add=True)
    pltpu.sync_copy(sz, accz.at[ridx], add=True)
    pltpu.sync_copy(rx, accn.at[ridx], add=True)
    return 0
  lax.fori_loop(0, NCH, chunk, 0)
  plsc.subcore_barrier()

  for comp, acc in enumerate((accx, accy, accz, accn)):
    def wb(k, _, comp=comp, acc=acc):
      c = sid + NS * k
      @pl.when(c < NWB)
      def _():
        pltpu.sync_copy(acc.at[pl.ds(c * WB, WB)], sx.at[pl.ds(0, WB)])
        pltpu.sync_copy(sx.at[pl.ds(0, WB)],
                        out.at[pl.ds(c * 8 * WB + (cid * 4 + comp) * WB, WB)])
      return 0
    lax.fori_loop(0, WB_IT, wb, 0)


@functools.cache
def _sc_edge_kernel():
  return pl.kernel(
    _sc_body,
    out_type=jax.ShapeDtypeStruct((NWB * 2 * 4 * WB,), jnp.float32),
    mesh=plsc.VectorSubcoreMesh(core_axis_name="c", subcore_axis_name="s",
                                num_cores=NC, num_subcores=NS),
    scratch_types=[
        pltpu.VMEM_SHARED((N,), jnp.float32),
        pltpu.VMEM_SHARED((N,), jnp.float32),
        pltpu.VMEM_SHARED((N,), jnp.float32),
        pltpu.VMEM_SHARED((N,), jnp.float32),
        pltpu.VMEM_SHARED((N,), jnp.float32),
        pltpu.VMEM_SHARED((N,), jnp.float32),
        pltpu.VMEM_SHARED((N,), jnp.float32),
        pltpu.VMEM((C,), jnp.int32),
        pltpu.VMEM((C,), jnp.int32),
        pltpu.VMEM((C,), jnp.float32),
        pltpu.VMEM((C,), jnp.float32),
        pltpu.VMEM((C,), jnp.float32),
        pltpu.VMEM((C,), jnp.float32),
        pltpu.VMEM((C,), jnp.float32),
        pltpu.VMEM((C,), jnp.float32),
        pltpu.VMEM((C,), jnp.float32),
    ],
  )

def _tc_a_body(pos_ref, emb_ref, pt_ref, out_ref):
  pos = pos_ref[...]
  emb = emb_ref[...]
  pt = pt_ref[...]

  mrp = pos[:, 15:18]
  vel = pos[:, 3:18] - pos[:, 0:15]
  nvel = (vel - VEL_MEAN) / VEL_STD
  relp = pos[:, 0:15] - jnp.concatenate([mrp] * 5, axis=1)
  db = jnp.concatenate([mrp - jnp.float32(0.1), jnp.float32(0.9) - mrp],
                       axis=1)
  ncdb = jnp.clip(db * INV_R, -1.0, 1.0)

  oh = (pt == lax.broadcasted_iota(jnp.int32, (BN, 9), 1)).astype(jnp.float32)
  temb = jnp.dot(oh, emb, preferred_element_type=jnp.float32)

  out_ref[...] = jnp.concatenate([nvel, relp, ncdb, temb], axis=1)


_tc_a_kernel = pl.pallas_call(
    _tc_a_body,
    grid=(NB,),
    in_specs=[
        pl.BlockSpec((BN, 18), lambda i: (i, 0)),
        pl.BlockSpec((9, 16), lambda i: (0, 0)),
        pl.BlockSpec((BN, 1), lambda i: (i, 0)),
    ],
    out_specs=pl.BlockSpec((BN, 52), lambda i: (i, 0)),
    out_shape=jax.ShapeDtypeStruct((N, 52), jnp.float32),
)


def _tc_b_body(pos_ref, agg_ref, a_ref, out_ref):
  pT = pos_ref[0]
  pr = agg_ref[0]

  v = pT[6:9] - pT[3:6]
  vp = pT[3:6] - pT[0:3]
  a = v - vp
  eps = jnp.float32(1e-8)
  e1 = v / (jnp.sqrt(jnp.sum(v * v, axis=0, keepdims=True)) + eps)
  u2 = a - jnp.sum(e1 * a, axis=0, keepdims=True) * e1
  e2 = u2 / (jnp.sqrt(jnp.sum(u2 * u2, axis=0, keepdims=True)) + eps)
  e3 = jnp.concatenate([
      e1[1:2] * e2[2:3] - e1[2:3] * e2[1:2],
      e1[2:3] * e2[0:1] - e1[0:1] * e2[2:3],
      e1[0:1] * e2[1:2] - e1[1:2] * e2[0:1],
  ], axis=0)
  rT = jnp.concatenate([
      e1[0:1], e2[0:1], e3[0:1],
      e1[1:2], e2[1:2], e3[1:2],
      e1[2:3], e2[2:3], e3[2:3],
  ], axis=0)

  big = jnp.concatenate([pr, rT], axis=0)
  r = lax.broadcasted_iota(jnp.int32, (17, 13), 0)
  c = lax.broadcasted_iota(jnp.int32, (17, 13), 1)
  m = (((r < 8) & (c < 4) & (r % 4 == c))
       | ((r >= 8) & (c >= 4) & (r - 8 == c - 4))).astype(jnp.float32)
  stripe = lax.dot_general(big, m,
                           dimension_numbers=(((0,), (0,)), ((), ())),
                           preferred_element_type=jnp.float32)
  out_ref[...] = jnp.concatenate([a_ref[...], stripe], axis=1)


_tc_b_kernel = pl.pallas_call(
    _tc_b_body,
    grid=(NB,),
    in_specs=[
        pl.BlockSpec((1, 9, BN), lambda i: (i, 0, 0)),
        pl.BlockSpec((1, 8, BN), lambda i: (i, 0, 0)),
        pl.BlockSpec((BN, 52), lambda i: (i, 0)),
    ],
    out_specs=pl.BlockSpec((BN, 65), lambda i: (i, 0)),
    out_shape=jax.ShapeDtypeStruct((N, 65), jnp.float32),
)


def kernel(position_sequence, type_embedding, particle_types, edge_index):
  mrp = position_sequence[:, -1]
  xs = mrp[:, 0]
  ys = mrp[:, 1]
  zs = mrp[:, 2]
  partials = _sc_edge_kernel()(xs, ys, zs, edge_index[0], edge_index[1])
  partials = partials.reshape(NWB, 8, WB)
  pos18 = position_sequence.reshape(N, 18)
  out_a = _tc_a_kernel(pos18, type_embedding, particle_types.reshape(N, 1))
  pos_t9 = (position_sequence[:, 3:, :]
            .reshape(NB, BN, 9).transpose(0, 2, 1))
  return _tc_b_kernel(pos_t9, partials, out_a)


# speedup vs baseline: 1.3551x; 1.3111x over previous
import functools

import jax
import jax.numpy as jnp
import numpy as np
from jax import lax
from jax.experimental import pallas as pl
from jax.experimental.pallas import tpu as pltpu
from jax.experimental.pallas import tpu_sc as plsc

N = 100000
E = 3200000
RADIUS = 0.015
INV_R = np.float32(1.0 / RADIUS)

NC = 2
NS = 16
NW = NC * NS
TE = E // NW
C = 4000
NCH = TE // C
NCHN = N // C
STAGE_IT = (NCHN + NS - 1) // NS

BN = 2000
NB = N // BN
WB = BN
NWB = N // WB
WB_IT = (NWB + NS - 1) // NS
VEL_MEAN = np.float32(0.0)
VEL_STD = np.float32(1.0)


def _rsqrt_newton(s):
  i = lax.bitcast_convert_type(s, jnp.int32)
  i = jnp.int32(0x5F3759DF) - lax.shift_right_logical(i, 1)
  y = lax.bitcast_convert_type(i, jnp.float32)
  for _ in range(3):
    y = y * (jnp.float32(1.5) - jnp.float32(0.5) * s * y * y)
  return y


def _sc_body(xs, ys, zs, snd, rcv, out,
             shx, shy, shz, accx, accy, accz, accn,
             sidx, ridx, sx, sy, sz, rx, ry, rz, dx, dy, dz, dn, zbuf):
  cid = lax.axis_index("c")
  sid = lax.axis_index("s")
  wid = cid * NS + sid

  def zb(i, _):
    zbuf[pl.ds(i * 16, 16)] = jnp.zeros((16,), jnp.float32)
    return 0
  lax.fori_loop(0, C // 16, zb, 0)

  for src, dst in ((xs, shx), (ys, shy), (zs, shz)):
    def stage(k, _, src=src, dst=dst):
      c = sid + NS * k
      @pl.when(c < NCHN)
      def _():
        pltpu.sync_copy(src.at[pl.ds(c * C, C)], sx)
        pltpu.sync_copy(sx, dst.at[pl.ds(c * C, C)])
      return 0
    lax.fori_loop(0, STAGE_IT, stage, 0)
  for acc in (accx, accy, accz, accn):
    def zero(k, _, acc=acc):
      c = sid + NS * k
      @pl.when(c < NCHN)
      def _():
        pltpu.sync_copy(zbuf, acc.at[pl.ds(c * C, C)])
      return 0
    lax.fori_loop(0, STAGE_IT, zero, 0)
  plsc.subcore_barrier()

  def chunk(g, _):
    base = wid * TE + g * C
    pltpu.sync_copy(snd.at[pl.ds(base, C)], sidx)
    pltpu.sync_copy(rcv.at[pl.ds(base, C)], ridx)
    pltpu.sync_copy(shx.at[sidx], sx)
    pltpu.sync_copy(shy.at[sidx], sy)
    pltpu.sync_copy(shz.at[sidx], sz)
    pltpu.sync_copy(shx.at[ridx], rx)
    pltpu.sync_copy(shy.at[ridx], ry)
    pltpu.sync_copy(shz.at[ridx], rz)

    def compute(i, _):
      s = pl.ds(i * 16, 16)
      vdx = (sx[s] - rx[s]) * INV_R
      vdy = (sy[s] - ry[s]) * INV_R
      vdz = (sz[s] - rz[s]) * INV_R
      ss = vdx * vdx + vdy * vdy + vdz * vdz
      vn = ss * _rsqrt_newton(ss)
      dx[s] = vdx
      dy[s] = vdy
      dz[s] = vdz
      dn[s] = vn
      return 0
    lax.fori_loop(0, C // 16, compute, 0)

    pltpu.sync_copy(dx, accx.at[ridx], add=True)
    pltpu.sync_copy(dy, accy.at[ridx], add=True)
    pltpu.sync_copy(dz, accz.at[ridx], add=True)
    pltpu.sync_copy(dn, accn.at[ridx], add=True)
    return 0
  lax.fori_loop(0, NCH, chunk, 0)
  plsc.subcore_barrier()

  for comp, acc in enumerate((accx, accy, accz, accn)):
    def wb(k, _, comp=comp, acc=acc):
      c = sid + NS * k
      @pl.when(c < NWB)
      def _():
        pltpu.sync_copy(acc.at[pl.ds(c * WB, WB)], sx.at[pl.ds(0, WB)])
        pltpu.sync_copy(sx.at[pl.ds(0, WB)],
                        out.at[pl.ds(c * 8 * WB + (cid * 4 + comp) * WB, WB)])
      return 0
    lax.fori_loop(0, WB_IT, wb, 0)


@functools.cache
def _sc_edge_kernel():
  return pl.kernel(
    _sc_body,
    out_type=jax.ShapeDtypeStruct((NWB * 2 * 4 * WB,), jnp.float32),
    mesh=plsc.VectorSubcoreMesh(core_axis_name="c", subcore_axis_name="s",
                                num_cores=NC, num_subcores=NS),
    scratch_types=[
        pltpu.VMEM_SHARED((N,), jnp.float32),
        pltpu.VMEM_SHARED((N,), jnp.float32),
        pltpu.VMEM_SHARED((N,), jnp.float32),
        pltpu.VMEM_SHARED((N,), jnp.float32),
        pltpu.VMEM_SHARED((N,), jnp.float32),
        pltpu.VMEM_SHARED((N,), jnp.float32),
        pltpu.VMEM_SHARED((N,), jnp.float32),
        pltpu.VMEM((C,), jnp.int32),
        pltpu.VMEM((C,), jnp.int32),
        pltpu.VMEM((C,), jnp.float32),
        pltpu.VMEM((C,), jnp.float32),
        pltpu.VMEM((C,), jnp.float32),
        pltpu.VMEM((C,), jnp.float32),
        pltpu.VMEM((C,), jnp.float32),
        pltpu.VMEM((C,), jnp.float32),
        pltpu.VMEM((C,), jnp.float32),
        pltpu.VMEM((C,), jnp.float32),
        pltpu.VMEM((C,), jnp.float32),
        pltpu.VMEM((C,), jnp.float32),
        pltpu.VMEM((C,), jnp.float32),
    ],
  )

def _tc_a_body(pos_ref, emb_ref, pt_ref, out_ref):
  pos = pos_ref[...]
  emb = emb_ref[...]
  pt = pt_ref[...]

  mrp = pos[:, 15:18]
  vel = pos[:, 3:18] - pos[:, 0:15]
  nvel = (vel - VEL_MEAN) / VEL_STD
  relp = pos[:, 0:15] - jnp.concatenate([mrp] * 5, axis=1)
  db = jnp.concatenate([mrp - jnp.float32(0.1), jnp.float32(0.9) - mrp],
                       axis=1)
  ncdb = jnp.clip(db * INV_R, -1.0, 1.0)

  oh = (pt == lax.broadcasted_iota(jnp.int32, (BN, 9), 1)).astype(jnp.float32)
  temb = jnp.dot(oh, emb, preferred_element_type=jnp.float32)

  out_ref[...] = jnp.concatenate([nvel, relp, ncdb, temb], axis=1)


_tc_a_kernel = pl.pallas_call(
    _tc_a_body,
    grid=(NB,),
    in_specs=[
        pl.BlockSpec((BN, 18), lambda i: (i, 0)),
        pl.BlockSpec((9, 16), lambda i: (0, 0)),
        pl.BlockSpec((BN, 1), lambda i: (i, 0)),
    ],
    out_specs=pl.BlockSpec((BN, 52), lambda i: (i, 0)),
    out_shape=jax.ShapeDtypeStruct((N, 52), jnp.float32),
)


def _tc_b_body(pos_ref, agg_ref, a_ref, out_ref):
  pT = pos_ref[0]
  pr = agg_ref[0]

  v = pT[6:9] - pT[3:6]
  vp = pT[3:6] - pT[0:3]
  a = v - vp
  eps = jnp.float32(1e-8)
  e1 = v / (jnp.sqrt(jnp.sum(v * v, axis=0, keepdims=True)) + eps)
  u2 = a - jnp.sum(e1 * a, axis=0, keepdims=True) * e1
  e2 = u2 / (jnp.sqrt(jnp.sum(u2 * u2, axis=0, keepdims=True)) + eps)
  e3 = jnp.concatenate([
      e1[1:2] * e2[2:3] - e1[2:3] * e2[1:2],
      e1[2:3] * e2[0:1] - e1[0:1] * e2[2:3],
      e1[0:1] * e2[1:2] - e1[1:2] * e2[0:1],
  ], axis=0)
  rT = jnp.concatenate([
      e1[0:1], e2[0:1], e3[0:1],
      e1[1:2], e2[1:2], e3[1:2],
      e1[2:3], e2[2:3], e3[2:3],
  ], axis=0)

  big = jnp.concatenate([pr, rT], axis=0)
  r = lax.broadcasted_iota(jnp.int32, (17, 13), 0)
  c = lax.broadcasted_iota(jnp.int32, (17, 13), 1)
  m = (((r < 8) & (c < 4) & (r % 4 == c))
       | ((r >= 8) & (c >= 4) & (r - 8 == c - 4))).astype(jnp.float32)
  stripe = lax.dot_general(big, m,
                           dimension_numbers=(((0,), (0,)), ((), ())),
                           preferred_element_type=jnp.float32)
  out_ref[...] = jnp.concatenate([a_ref[...], stripe], axis=1)


_tc_b_kernel = pl.pallas_call(
    _tc_b_body,
    grid=(NB,),
    in_specs=[
        pl.BlockSpec((1, 9, BN), lambda i: (i, 0, 0)),
        pl.BlockSpec((1, 8, BN), lambda i: (i, 0, 0)),
        pl.BlockSpec((BN, 52), lambda i: (i, 0)),
    ],
    out_specs=pl.BlockSpec((BN, 65), lambda i: (i, 0)),
    out_shape=jax.ShapeDtypeStruct((N, 65), jnp.float32),
)


def kernel(position_sequence, type_embedding, particle_types, edge_index):
  mrp = position_sequence[:, -1]
  xs = mrp[:, 0]
  ys = mrp[:, 1]
  zs = mrp[:, 2]
  partials = _sc_edge_kernel()(xs, ys, zs, edge_index[0], edge_index[1])
  partials = partials.reshape(NWB, 8, WB)
  pos18 = position_sequence.reshape(N, 18)
  out_a = _tc_a_kernel(pos18, type_embedding, particle_types.reshape(N, 1))
  pos_t9 = (position_sequence[:, 3:, :]
            .reshape(NB, BN, 9).transpose(0, 2, 1))
  return _tc_b_kernel(pos_t9, partials, out_a)


# speedup vs baseline: 1.5193x; 1.1211x over previous
import functools

import jax
import jax.numpy as jnp
import numpy as np
from jax import lax
from jax.experimental import pallas as pl
from jax.experimental.pallas import tpu as pltpu
from jax.experimental.pallas import tpu_sc as plsc

N = 100000
E = 3200000
RADIUS = 0.015
INV_R = np.float32(1.0 / RADIUS)

NC = 2
NS = 16
NW = NC * NS
TE = E // NW
C = 4000
NCH = TE // C
NCHN = N // C
STAGE_IT = (NCHN + NS - 1) // NS

BN = 2000
NB = N // BN
WB = BN
NWB = N // WB
WB_IT = (NWB + NS - 1) // NS
VEL_MEAN = np.float32(0.0)
VEL_STD = np.float32(1.0)


def _rsqrt_newton(s):
  i = lax.bitcast_convert_type(s, jnp.int32)
  i = jnp.int32(0x5F3759DF) - lax.shift_right_logical(i, 1)
  y = lax.bitcast_convert_type(i, jnp.float32)
  for _ in range(3):
    y = y * (jnp.float32(1.5) - jnp.float32(0.5) * s * y * y)
  return y


def _sc_body(xs, ys, zs, snd, rcv, out,
             shx, shy, shz, accx, accy, accz, accn,
             si0, si1, ri0, ri1,
             sx0, sy0, sz0, rx0, ry0, rz0,
             sx1, sy1, sz1, rx1, ry1, rz1,
             dx, dy, dz, dn, zbuf, semg0, semg1, sems2):
  sx = sx0
  cid = lax.axis_index("c")
  sid = lax.axis_index("s")
  wid = cid * NS + sid

  def zb(i, _):
    zbuf[pl.ds(i * 16, 16)] = jnp.zeros((16,), jnp.float32)
    return 0
  lax.fori_loop(0, C // 16, zb, 0)

  for src, dst in ((xs, shx), (ys, shy), (zs, shz)):
    def stage(k, _, src=src, dst=dst):
      c = sid + NS * k
      @pl.when(c < NCHN)
      def _():
        pltpu.sync_copy(src.at[pl.ds(c * C, C)], sx)
        pltpu.sync_copy(sx, dst.at[pl.ds(c * C, C)])
      return 0
    lax.fori_loop(0, STAGE_IT, stage, 0)
  for acc in (accx, accy, accz, accn):
    def zero(k, _, acc=acc):
      c = sid + NS * k
      @pl.when(c < NCHN)
      def _():
        pltpu.sync_copy(zbuf, acc.at[pl.ds(c * C, C)])
      return 0
    lax.fori_loop(0, STAGE_IT, zero, 0)
  plsc.subcore_barrier()

  sidx2 = (si0, si1)
  ridx2 = (ri0, ri1)
  gsets = ((sx0, sy0, sz0), (sx1, sy1, sz1))
  g2sets = ((rx0, ry0, rz0), (rx1, ry1, rz1))
  sems = (semg0, semg1)

  def issue_gathers(b, g):
    base = wid * TE + g * C
    pltpu.sync_copy(snd.at[pl.ds(base, C)], sidx2[b])
    pltpu.sync_copy(rcv.at[pl.ds(base, C)], ridx2[b])
    for tab, dst in zip((shx, shy, shz), gsets[b]):
      pltpu.async_copy(tab.at[sidx2[b]], dst, sems[b])
    for tab, dst in zip((shx, shy, shz), g2sets[b]):
      pltpu.async_copy(tab.at[ridx2[b]], dst, sems[b])

  def wait_gathers(b):
    for dst in gsets[b] + g2sets[b]:
      pltpu.make_async_copy(snd.at[pl.ds(0, C)], dst, sems[b]).wait()

  def wait_scatters():
    for src in (dx, dy, dz, dn):
      pltpu.make_async_copy(snd.at[pl.ds(0, C)], src, sems2).wait()

  def body(b, g):
    wait_gathers(b)
    @pl.when(g > 0)
    def _():
      wait_scatters()
    @pl.when(g + 1 < NCH)
    def _():
      issue_gathers(1 - b, g + 1)
    gx, gy, gz = gsets[b]
    hx, hy, hz = g2sets[b]

    def compute(i, _):
      s = pl.ds(i * 16, 16)
      vdx = (gx[s] - hx[s]) * INV_R
      vdy = (gy[s] - hy[s]) * INV_R
      vdz = (gz[s] - hz[s]) * INV_R
      ss = vdx * vdx + vdy * vdy + vdz * vdz
      vn = ss * _rsqrt_newton(ss)
      dx[s] = vdx
      dy[s] = vdy
      dz[s] = vdz
      dn[s] = vn
      return 0
    lax.fori_loop(0, C // 16, compute, 0)

    pltpu.async_copy(dx, accx.at[ridx2[b]], sems2, add=True)
    pltpu.async_copy(dy, accy.at[ridx2[b]], sems2, add=True)
    pltpu.async_copy(dz, accz.at[ridx2[b]], sems2, add=True)
    pltpu.async_copy(dn, accn.at[ridx2[b]], sems2, add=True)

  issue_gathers(0, 0)

  def chunk(g, _):
    for b in (0, 1):
      @pl.when(g % 2 == b)
      def _(b=b):
        body(b, g)
    return 0
  lax.fori_loop(0, NCH, chunk, 0)
  wait_scatters()
  plsc.subcore_barrier()

  for comp, acc in enumerate((accx, accy, accz, accn)):
    def wb(k, _, comp=comp, acc=acc):
      c = sid + NS * k
      @pl.when(c < NWB)
      def _():
        pltpu.sync_copy(acc.at[pl.ds(c * WB, WB)], sx.at[pl.ds(0, WB)])
        pltpu.sync_copy(sx.at[pl.ds(0, WB)],
                        out.at[pl.ds(c * 8 * WB + (cid * 4 + comp) * WB, WB)])
      return 0
    lax.fori_loop(0, WB_IT, wb, 0)


@functools.cache
def _sc_edge_kernel():
  return pl.kernel(
    _sc_body,
    out_type=jax.ShapeDtypeStruct((NWB * 2 * 4 * WB,), jnp.float32),
    mesh=plsc.VectorSubcoreMesh(core_axis_name="c", subcore_axis_name="s",
                                num_cores=NC, num_subcores=NS),
    scratch_types=[
        pltpu.VMEM_SHARED((N,), jnp.float32),
        pltpu.VMEM_SHARED((N,), jnp.float32),
        pltpu.VMEM_SHARED((N,), jnp.float32),
        pltpu.VMEM_SHARED((N,), jnp.float32),
        pltpu.VMEM_SHARED((N,), jnp.float32),
        pltpu.VMEM_SHARED((N,), jnp.float32),
        pltpu.VMEM_SHARED((N,), jnp.float32),
        pltpu.VMEM((C,), jnp.int32),
        pltpu.VMEM((C,), jnp.int32),
        pltpu.VMEM((C,), jnp.int32),
        pltpu.VMEM((C,), jnp.int32),
    ] + [pltpu.VMEM((C,), jnp.float32)] * 17 + [
        pltpu.SemaphoreType.DMA,
        pltpu.SemaphoreType.DMA,
        pltpu.SemaphoreType.DMA,
    ],
  )

def _tc_a_body(pos_ref, emb_ref, pt_ref, out_ref):
  pos = pos_ref[...]
  emb = emb_ref[...]
  pt = pt_ref[...]

  mrp = pos[:, 15:18]
  vel = pos[:, 3:18] - pos[:, 0:15]
  nvel = (vel - VEL_MEAN) / VEL_STD
  relp = pos[:, 0:15] - jnp.concatenate([mrp] * 5, axis=1)
  db = jnp.concatenate([mrp - jnp.float32(0.1), jnp.float32(0.9) - mrp],
                       axis=1)
  ncdb = jnp.clip(db * INV_R, -1.0, 1.0)

  oh = (pt == lax.broadcasted_iota(jnp.int32, (BN, 9), 1)).astype(jnp.float32)
  temb = jnp.dot(oh, emb, preferred_element_type=jnp.float32)

  out_ref[...] = jnp.concatenate([nvel, relp, ncdb, temb], axis=1)


_tc_a_kernel = pl.pallas_call(
    _tc_a_body,
    grid=(NB,),
    in_specs=[
        pl.BlockSpec((BN, 18), lambda i: (i, 0)),
        pl.BlockSpec((9, 16), lambda i: (0, 0)),
        pl.BlockSpec((BN, 1), lambda i: (i, 0)),
    ],
    out_specs=pl.BlockSpec((BN, 52), lambda i: (i, 0)),
    out_shape=jax.ShapeDtypeStruct((N, 52), jnp.float32),
)


def _tc_b_body(pos_ref, agg_ref, a_ref, out_ref):
  pT = pos_ref[0]
  pr = agg_ref[0]

  v = pT[6:9] - pT[3:6]
  vp = pT[3:6] - pT[0:3]
  a = v - vp
  eps = jnp.float32(1e-8)
  e1 = v / (jnp.sqrt(jnp.sum(v * v, axis=0, keepdims=True)) + eps)
  u2 = a - jnp.sum(e1 * a, axis=0, keepdims=True) * e1
  e2 = u2 / (jnp.sqrt(jnp.sum(u2 * u2, axis=0, keepdims=True)) + eps)
  e3 = jnp.concatenate([
      e1[1:2] * e2[2:3] - e1[2:3] * e2[1:2],
      e1[2:3] * e2[0:1] - e1[0:1] * e2[2:3],
      e1[0:1] * e2[1:2] - e1[1:2] * e2[0:1],
  ], axis=0)
  rT = jnp.concatenate([
      e1[0:1], e2[0:1], e3[0:1],
      e1[1:2], e2[1:2], e3[1:2],
      e1[2:3], e2[2:3], e3[2:3],
  ], axis=0)

  big = jnp.concatenate([pr, rT], axis=0)
  r = lax.broadcasted_iota(jnp.int32, (17, 13), 0)
  c = lax.broadcasted_iota(jnp.int32, (17, 13), 1)
  m = (((r < 8) & (c < 4) & (r % 4 == c))
       | ((r >= 8) & (c >= 4) & (r - 8 == c - 4))).astype(jnp.float32)
  stripe = lax.dot_general(big, m,
                           dimension_numbers=(((0,), (0,)), ((), ())),
                           preferred_element_type=jnp.float32)
  out_ref[...] = jnp.concatenate([a_ref[...], stripe], axis=1)


_tc_b_kernel = pl.pallas_call(
    _tc_b_body,
    grid=(NB,),
    in_specs=[
        pl.BlockSpec((1, 9, BN), lambda i: (i, 0, 0)),
        pl.BlockSpec((1, 8, BN), lambda i: (i, 0, 0)),
        pl.BlockSpec((BN, 52), lambda i: (i, 0)),
    ],
    out_specs=pl.BlockSpec((BN, 65), lambda i: (i, 0)),
    out_shape=jax.ShapeDtypeStruct((N, 65), jnp.float32),
)


def kernel(position_sequence, type_embedding, particle_types, edge_index):
  mrp = position_sequence[:, -1]
  xs = mrp[:, 0]
  ys = mrp[:, 1]
  zs = mrp[:, 2]
  partials = _sc_edge_kernel()(xs, ys, zs, edge_index[0], edge_index[1])
  partials = partials.reshape(NWB, 8, WB)
  pos18 = position_sequence.reshape(N, 18)
  out_a = _tc_a_kernel(pos18, type_embedding, particle_types.reshape(N, 1))
  pos_t9 = (position_sequence[:, 3:, :]
            .reshape(NB, BN, 9).transpose(0, 2, 1))
  return _tc_b_kernel(pos_t9, partials, out_a)


# speedup vs baseline: 1.5209x; 1.0010x over previous
import functools

import jax
import jax.numpy as jnp
import numpy as np
from jax import lax
from jax.experimental import pallas as pl
from jax.experimental.pallas import tpu as pltpu
from jax.experimental.pallas import tpu_sc as plsc

N = 100000
E = 3200000
RADIUS = 0.015
INV_R = np.float32(1.0 / RADIUS)

NC = 2
NS = 16
NW = NC * NS
TE = E // NW
C = 4000
NCH = TE // C
NCHN = N // C
STAGE_IT = (NCHN + NS - 1) // NS

BN = 2000
NB = N // BN
WB = BN
NWB = N // WB
WB_IT = (NWB + NS - 1) // NS
VEL_MEAN = np.float32(0.0)
VEL_STD = np.float32(1.0)


def _rsqrt_newton(s):
  i = lax.bitcast_convert_type(s, jnp.int32)
  i = jnp.int32(0x5F3759DF) - lax.shift_right_logical(i, 1)
  y = lax.bitcast_convert_type(i, jnp.float32)
  for _ in range(3):
    y = y * (jnp.float32(1.5) - jnp.float32(0.5) * s * y * y)
  return y


def _sc_body(xs, ys, zs, snd, rcv, out,
             shx, shy, shz, accx, accy, accz, accn,
             si0, si1, ri0, ri1,
             sx0, sy0, sz0, rx0, ry0, rz0,
             sx1, sy1, sz1, rx1, ry1, rz1,
             dx, dy, dz, dn, zbuf, semg0, semg1, sems2):
  sx = sx0
  cid = lax.axis_index("c")
  sid = lax.axis_index("s")
  wid = cid * NS + sid

  def zb(i, _):
    zbuf[pl.ds(i * 16, 16)] = jnp.zeros((16,), jnp.float32)
    return 0
  lax.fori_loop(0, C // 16, zb, 0)

  for src, dst in ((xs, shx), (ys, shy), (zs, shz)):
    def stage(k, _, src=src, dst=dst):
      c = sid + NS * k
      @pl.when(c < NCHN)
      def _():
        pltpu.sync_copy(src.at[pl.ds(c * C, C)], sx)
        pltpu.sync_copy(sx, dst.at[pl.ds(c * C, C)])
      return 0
    lax.fori_loop(0, STAGE_IT, stage, 0)
  for acc in (accx, accy, accz, accn):
    def zero(k, _, acc=acc):
      c = sid + NS * k
      @pl.when(c < NCHN)
      def _():
        pltpu.sync_copy(zbuf, acc.at[pl.ds(c * C, C)])
      return 0
    lax.fori_loop(0, STAGE_IT, zero, 0)
  plsc.subcore_barrier()

  sidx2 = (si0, si1)
  ridx2 = (ri0, ri1)
  gsets = ((sx0, sy0, sz0), (sx1, sy1, sz1))
  g2sets = ((rx0, ry0, rz0), (rx1, ry1, rz1))
  sems = (semg0, semg1)

  def issue_gathers(b, g):
    base = wid * TE + g * C
    pltpu.sync_copy(snd.at[pl.ds(base, C)], sidx2[b])
    pltpu.sync_copy(rcv.at[pl.ds(base, C)], ridx2[b])
    for tab, dst in zip((shx, shy, shz), gsets[b]):
      pltpu.async_copy(tab.at[sidx2[b]], dst, sems[b])
    for tab, dst in zip((shx, shy, shz), g2sets[b]):
      pltpu.async_copy(tab.at[ridx2[b]], dst, sems[b])

  def wait_gathers(b):
    for dst in gsets[b] + g2sets[b]:
      pltpu.make_async_copy(snd.at[pl.ds(0, C)], dst, sems[b]).wait()

  def wait_scatters():
    for src in (dx, dy, dz, dn):
      pltpu.make_async_copy(snd.at[pl.ds(0, C)], src, sems2).wait()

  def body(b, g):
    wait_gathers(b)
    @pl.when(g > 0)
    def _():
      wait_scatters()
    @pl.when(g + 1 < NCH)
    def _():
      issue_gathers(1 - b, g + 1)
    gx, gy, gz = gsets[b]
    hx, hy, hz = g2sets[b]

    def compute(i, _):
      s = pl.ds(i * 16, 16)
      vdx = (gx[s] - hx[s]) * INV_R
      vdy = (gy[s] - hy[s]) * INV_R
      vdz = (gz[s] - hz[s]) * INV_R
      ss = vdx * vdx + vdy * vdy + vdz * vdz
      vn = ss * _rsqrt_newton(ss)
      dx[s] = vdx
      dy[s] = vdy
      dz[s] = vdz
      dn[s] = vn
      return 0
    lax.fori_loop(0, C // 16, compute, 0)

    pltpu.async_copy(dx, accx.at[ridx2[b]], sems2, add=True)
    pltpu.async_copy(dy, accy.at[ridx2[b]], sems2, add=True)
    pltpu.async_copy(dz, accz.at[ridx2[b]], sems2, add=True)
    pltpu.async_copy(dn, accn.at[ridx2[b]], sems2, add=True)

  issue_gathers(0, 0)

  def chunk(g, _):
    for b in (0, 1):
      @pl.when(g % 2 == b)
      def _(b=b):
        body(b, g)
    return 0
  lax.fori_loop(0, NCH, chunk, 0)
  wait_scatters()
  plsc.subcore_barrier()

  for comp, acc in enumerate((accx, accy, accz, accn)):
    def wb(k, _, comp=comp, acc=acc):
      c = sid + NS * k
      @pl.when(c < NWB)
      def _():
        pltpu.sync_copy(acc.at[pl.ds(c * WB, WB)], sx.at[pl.ds(0, WB)])
        pltpu.sync_copy(sx.at[pl.ds(0, WB)],
                        out.at[pl.ds(c * 8 * WB + (cid * 4 + comp) * WB, WB)])
      return 0
    lax.fori_loop(0, WB_IT, wb, 0)


@functools.cache
def _sc_edge_kernel():
  return pl.kernel(
    _sc_body,
    out_type=jax.ShapeDtypeStruct((NWB * 2 * 4 * WB,), jnp.float32),
    mesh=plsc.VectorSubcoreMesh(core_axis_name="c", subcore_axis_name="s",
                                num_cores=NC, num_subcores=NS),
    scratch_types=[
        pltpu.VMEM_SHARED((N,), jnp.float32),
        pltpu.VMEM_SHARED((N,), jnp.float32),
        pltpu.VMEM_SHARED((N,), jnp.float32),
        pltpu.VMEM_SHARED((N,), jnp.float32),
        pltpu.VMEM_SHARED((N,), jnp.float32),
        pltpu.VMEM_SHARED((N,), jnp.float32),
        pltpu.VMEM_SHARED((N,), jnp.float32),
        pltpu.VMEM((C,), jnp.int32),
        pltpu.VMEM((C,), jnp.int32),
        pltpu.VMEM((C,), jnp.int32),
        pltpu.VMEM((C,), jnp.int32),
    ] + [pltpu.VMEM((C,), jnp.float32)] * 17 + [
        pltpu.SemaphoreType.DMA,
        pltpu.SemaphoreType.DMA,
        pltpu.SemaphoreType.DMA,
    ],
  )

def _tc_a_body(pos_ref, emb_ref, pt_ref, pt9_ref, out_ref):
  del pt9_ref
  pos = pos_ref[...]
  emb = emb_ref[...]
  pt = pt_ref[...]

  mrp = pos[:, 15:18]
  vel = pos[:, 3:18] - pos[:, 0:15]
  nvel = (vel - VEL_MEAN) / VEL_STD
  relp = pos[:, 0:15] - jnp.concatenate([mrp] * 5, axis=1)
  db = jnp.concatenate([mrp - jnp.float32(0.1), jnp.float32(0.9) - mrp],
                       axis=1)
  ncdb = jnp.clip(db * INV_R, -1.0, 1.0)

  oh = (pt == lax.broadcasted_iota(jnp.int32, (BN, 9), 1)).astype(jnp.float32)
  temb = jnp.dot(oh, emb, preferred_element_type=jnp.float32)

  out_ref[...] = jnp.concatenate([nvel, relp, ncdb, temb], axis=1)


_tc_a_kernel = pl.pallas_call(
    _tc_a_body,
    grid=(NB,),
    in_specs=[
        pl.BlockSpec((BN, 18), lambda i: (i, 0)),
        pl.BlockSpec((9, 16), lambda i: (0, 0)),
        pl.BlockSpec((BN, 1), lambda i: (i, 0)),
        pl.BlockSpec((1, 9, BN), lambda i: (i, 0, 0)),
    ],
    out_specs=pl.BlockSpec((BN, 52), lambda i: (i, 0)),
    out_shape=jax.ShapeDtypeStruct((N, 52), jnp.float32),
)


def _tc_b_body(pos_ref, agg_ref, a_ref, out_ref):
  pT = pos_ref[0]
  pr = agg_ref[0]

  v = pT[6:9] - pT[3:6]
  vp = pT[3:6] - pT[0:3]
  a = v - vp
  eps = jnp.float32(1e-8)
  e1 = v / (jnp.sqrt(jnp.sum(v * v, axis=0, keepdims=True)) + eps)
  u2 = a - jnp.sum(e1 * a, axis=0, keepdims=True) * e1
  e2 = u2 / (jnp.sqrt(jnp.sum(u2 * u2, axis=0, keepdims=True)) + eps)
  e3 = jnp.concatenate([
      e1[1:2] * e2[2:3] - e1[2:3] * e2[1:2],
      e1[2:3] * e2[0:1] - e1[0:1] * e2[2:3],
      e1[0:1] * e2[1:2] - e1[1:2] * e2[0:1],
  ], axis=0)
  rT = jnp.concatenate([
      e1[0:1], e2[0:1], e3[0:1],
      e1[1:2], e2[1:2], e3[1:2],
      e1[2:3], e2[2:3], e3[2:3],
  ], axis=0)

  big = jnp.concatenate([pr, rT], axis=0)
  r = lax.broadcasted_iota(jnp.int32, (17, 13), 0)
  c = lax.broadcasted_iota(jnp.int32, (17, 13), 1)
  m = (((r < 8) & (c < 4) & (r % 4 == c))
       | ((r >= 8) & (c >= 4) & (r - 8 == c - 4))).astype(jnp.float32)
  stripe = lax.dot_general(big, m,
                           dimension_numbers=(((0,), (0,)), ((), ())),
                           preferred_element_type=jnp.float32)
  out_ref[...] = jnp.concatenate([a_ref[...], stripe], axis=1)


_tc_b_kernel = pl.pallas_call(
    _tc_b_body,
    grid=(NB,),
    in_specs=[
        pl.BlockSpec((1, 9, BN), lambda i: (i, 0, 0)),
        pl.BlockSpec((1, 8, BN), lambda i: (i, 0, 0)),
        pl.BlockSpec((BN, 52), lambda i: (i, 0)),
    ],
    out_specs=pl.BlockSpec((BN, 65), lambda i: (i, 0)),
    out_shape=jax.ShapeDtypeStruct((N, 65), jnp.float32),
)


def kernel(position_sequence, type_embedding, particle_types, edge_index):
  mrp = position_sequence[:, -1]
  xs = mrp[:, 0]
  ys = mrp[:, 1]
  zs = mrp[:, 2]
  partials = _sc_edge_kernel()(xs, ys, zs, edge_index[0], edge_index[1])
  partials = partials.reshape(NWB, 8, WB)
  pos18 = position_sequence.reshape(N, 18)
  pos_t9 = (position_sequence[:, 3:, :]
            .reshape(NB, BN, 9).transpose(0, 2, 1))
  out_a = _tc_a_kernel(pos18, type_embedding, particle_types.reshape(N, 1),
                       pos_t9)
  return _tc_b_kernel(pos_t9, partials, out_a)
